# baseline (speedup 1.0000x reference)
import jax
import jax.numpy as jnp
from jax.experimental import pallas as pl

_B = 4096
_L = 200
_D = 32
_BLK = 256


def _body(labelsT_ref, ewT_ref, out_ref):
    m = (labelsT_ref[...] != 0).astype(jnp.float32)
    out_ref[...] = jax.lax.dot(ewT_ref[...], m,
                               precision=jax.lax.Precision.HIGHEST,
                               preferred_element_type=jnp.float32)


def kernel(labels, weight):
    wflat = jax.lax.slice(weight, (1, 0), (1 + _L, _D)).reshape(_L * _D)
    row = jnp.arange(_L * _D, dtype=jnp.int32) // _D
    onehot = (row[:, None] == jnp.arange(_L, dtype=jnp.int32)[None, :])
    ewT = onehot.astype(jnp.float32) * wflat[:, None]
    labelsT = labels.T
    outT = pl.pallas_call(
        _body,
        grid=(_B // _BLK,),
        in_specs=[
            pl.BlockSpec((_L, _BLK), lambda i: (0, i)),
            pl.BlockSpec((_L * _D, _L), lambda i: (0, 0)),
        ],
        out_specs=pl.BlockSpec((_L * _D, _BLK), lambda i: (0, i)),
        out_shape=jax.ShapeDtypeStruct((_L * _D, _B), jnp.float32),
    )(labelsT, ewT)
    return outT.reshape(_L, _D, _B).transpose(2, 0, 1)


# speedup vs baseline: 1.7283x; 1.7283x over previous
import jax
import jax.numpy as jnp
from jax.experimental import pallas as pl

_B = 4096
_L = 200
_D = 32
_BLK = 512


def _body(labelsT_ref, ewhi_ref, ewlo_ref, out_ref):
    m = (labelsT_ref[...] != 0).astype(jnp.float32)
    hi = jax.lax.dot(ewhi_ref[...], m, preferred_element_type=jnp.float32)
    lo = jax.lax.dot(ewlo_ref[...], m, preferred_element_type=jnp.float32)
    out_ref[...] = hi + lo


def kernel(labels, weight):
    wflat = jax.lax.slice(weight, (1, 0), (1 + _L, _D)).reshape(_L * _D)
    row = jnp.arange(_L * _D, dtype=jnp.int32) // _D
    onehot = (row[:, None] == jnp.arange(_L, dtype=jnp.int32)[None, :])
    ewT = onehot.astype(jnp.float32) * wflat[:, None]
    ewhi = ewT.astype(jnp.bfloat16).astype(jnp.float32)
    ewlo = ewT - ewhi
    labelsT = labels.T
    outT = pl.pallas_call(
        _body,
        grid=(_B // _BLK,),
        in_specs=[
            pl.BlockSpec((_L, _BLK), lambda i: (0, i)),
            pl.BlockSpec((_L * _D, _L), lambda i: (0, 0)),
            pl.BlockSpec((_L * _D, _L), lambda i: (0, 0)),
        ],
        out_specs=pl.BlockSpec((_L * _D, _BLK), lambda i: (0, i)),
        out_shape=jax.ShapeDtypeStruct((_L * _D, _B), jnp.float32),
    )(labelsT, ewhi, ewlo)
    return outT.reshape(_L, _D, _B).transpose(2, 0, 1)


# speedup vs baseline: 2.1780x; 1.2602x over previous
import jax
import jax.numpy as jnp
from jax.experimental import pallas as pl

_B = 4096
_L = 200
_D = 32
_BLK = 512


def _body(labelsT_ref, w_ref, out_ref):
    m = labelsT_ref[...] != 0
    w = w_ref[...]
    out_ref[...] = jnp.where(m, w, 0.0)


def kernel(labels, weight):
    w3 = jax.lax.slice(weight, (1, 0), (1 + _L, _D)).reshape(_L, _D, 1)
    labelsT3 = labels.T.reshape(_L, 1, _B)
    outT = pl.pallas_call(
        _body,
        grid=(_B // _BLK,),
        in_specs=[
            pl.BlockSpec((_L, 1, _BLK), lambda i: (0, 0, i)),
            pl.BlockSpec((_L, _D, 1), lambda i: (0, 0, 0)),
        ],
        out_specs=pl.BlockSpec((_L, _D, _BLK), lambda i: (0, 0, i)),
        out_shape=jax.ShapeDtypeStruct((_L, _D, _B), jnp.float32),
    )(labelsT3, w3)
    return outT.transpose(2, 0, 1)
